# baseline (speedup 1.0000x reference)
import functools

import jax
import jax.numpy as jnp
from jax import lax
from jax.experimental import pallas as pl
from jax.experimental.pallas import tpu as pltpu
from jax.experimental.pallas import tpu_sc as plsc

_NC, _NS = 2, 16
_NW = _NC * _NS
_STREAM = 128
_TILE_B = 16384


def _sc_gather(user, rsid, users_table, rsids_table, B, L):
    b_per_w = B // _NW
    n_streams = b_per_w // _STREAM
    r_per_w = L // _NS

    mesh = plsc.VectorSubcoreMesh(core_axis_name="c", subcore_axis_name="s")

    @functools.partial(
        pl.kernel,
        out_type=(
            jax.ShapeDtypeStruct((B, L), jnp.float32),
            jax.ShapeDtypeStruct((L, L), jnp.float32),
        ),
        mesh=mesh,
        scratch_types=(
            [
                pltpu.VMEM((b_per_w,), jnp.int32),
                pltpu.VMEM((b_per_w, L), jnp.float32),
                pltpu.VMEM((r_per_w,), jnp.int32),
                pltpu.VMEM((r_per_w, L), jnp.float32),
                pltpu.SemaphoreType.DMA,
                pltpu.SemaphoreType.DMA,
            ]
            + [pltpu.SemaphoreType.DMA for _ in range(n_streams)]
        ),
    )
    def gather_kernel(user_hbm, rsid_hbm, utab_hbm, rtab_hbm, u_out, r_out,
                      uidx_v, urows_v, ridx_v, rrows_v, rsem, wsem, *gsems):
        wid = lax.axis_index("s") * _NC + lax.axis_index("c")
        base = wid * b_per_w

        pltpu.sync_copy(user_hbm.at[pl.ds(base, b_per_w)], uidx_v)
        gathers = [
            pltpu.async_copy(
                utab_hbm.at[uidx_v.at[pl.ds(j * _STREAM, _STREAM)]],
                urows_v.at[pl.ds(j * _STREAM, _STREAM)],
                gsems[j],
            )
            for j in range(n_streams)
        ]
        writes = []
        for j in range(n_streams):
            gathers[j].wait()
            writes.append(pltpu.async_copy(
                urows_v.at[pl.ds(j * _STREAM, _STREAM)],
                u_out.at[pl.ds(base + j * _STREAM, _STREAM)],
                wsem,
            ))

        @pl.when(wid < _NS)
        def _():
            pltpu.sync_copy(rsid_hbm.at[pl.ds(wid * r_per_w, r_per_w)],
                            ridx_v)
            pltpu.async_copy(rtab_hbm.at[ridx_v], rrows_v, rsem).wait()
            pltpu.sync_copy(rrows_v, r_out.at[pl.ds(wid * r_per_w, r_per_w)])

        for c in writes:
            c.wait()

    return gather_kernel(user, rsid, users_table, rsids_table)


def _tc_matmul(u, r, B, L):
    def mm_body(u_ref, r_ref, o_ref):
        o_ref[...] = jnp.dot(u_ref[...], r_ref[...],
                             preferred_element_type=jnp.float32)

    return pl.pallas_call(
        mm_body,
        grid=(B // _TILE_B,),
        in_specs=[
            pl.BlockSpec((_TILE_B, L), lambda i: (i, 0)),
            pl.BlockSpec((L, L), lambda i: (0, 0)),
        ],
        out_specs=pl.BlockSpec((_TILE_B, L), lambda i: (i, 0)),
        out_shape=jax.ShapeDtypeStruct((B, L), jnp.float32),
    )(u, r)


def kernel(user, rsid, users_table, rsids_table):
    B = user.shape[0]
    L = rsids_table.shape[1]
    u, r = _sc_gather(user, rsid, users_table, rsids_table, B, L)
    return _tc_matmul(u, r, B, L)


# speedup vs baseline: 1.0518x; 1.0518x over previous
import functools

import jax
import jax.numpy as jnp
from jax import lax
from jax.experimental import pallas as pl
from jax.experimental.pallas import tpu as pltpu
from jax.experimental.pallas import tpu_sc as plsc

_NC, _NS = 2, 16
_NW = _NC * _NS
_STREAM = 128
_TILE_B = 8192


def _sc_gather(user, rsid, users_table, rsids_table, B, L):
    b_per_w = B // _NW
    n_streams = b_per_w // _STREAM
    r_per_w = L // _NS

    mesh = plsc.VectorSubcoreMesh(core_axis_name="c", subcore_axis_name="s")

    @functools.partial(
        pl.kernel,
        out_type=(
            jax.ShapeDtypeStruct((B, L), jnp.float32),
            jax.ShapeDtypeStruct((L, L), jnp.float32),
        ),
        mesh=mesh,
        scratch_types=(
            [
                pltpu.VMEM((b_per_w,), jnp.int32),
                pltpu.VMEM((b_per_w, L), jnp.float32),
                pltpu.VMEM((r_per_w,), jnp.int32),
                pltpu.VMEM((r_per_w, L), jnp.float32),
                pltpu.SemaphoreType.DMA,
                pltpu.SemaphoreType.DMA,
            ]
            + [pltpu.SemaphoreType.DMA for _ in range(n_streams)]
        ),
    )
    def gather_kernel(user_hbm, rsid_hbm, utab_hbm, rtab_hbm, u_out, r_out,
                      uidx_v, urows_v, ridx_v, rrows_v, rsem, wsem, *gsems):
        wid = lax.axis_index("s") * _NC + lax.axis_index("c")
        base = wid * b_per_w

        pltpu.sync_copy(user_hbm.at[pl.ds(base, b_per_w)], uidx_v)
        gathers = [
            pltpu.async_copy(
                utab_hbm.at[uidx_v.at[pl.ds(j * _STREAM, _STREAM)]],
                urows_v.at[pl.ds(j * _STREAM, _STREAM)],
                gsems[j],
            )
            for j in range(n_streams)
        ]
        writes = []
        for j in range(n_streams):
            gathers[j].wait()
            writes.append(pltpu.async_copy(
                urows_v.at[pl.ds(j * _STREAM, _STREAM)],
                u_out.at[pl.ds(base + j * _STREAM, _STREAM)],
                wsem,
            ))

        @pl.when(wid < _NS)
        def _():
            pltpu.sync_copy(rsid_hbm.at[pl.ds(wid * r_per_w, r_per_w)],
                            ridx_v)
            pltpu.async_copy(rtab_hbm.at[ridx_v], rrows_v, rsem).wait()
            pltpu.sync_copy(rrows_v, r_out.at[pl.ds(wid * r_per_w, r_per_w)])

        for c in writes:
            c.wait()

    return gather_kernel(user, rsid, users_table, rsids_table)


def _tc_matmul(u, r, B, L):
    def mm_body(u_ref, r_ref, o_ref):
        o_ref[...] = jnp.dot(u_ref[...], r_ref[...],
                             preferred_element_type=jnp.float32)

    return pl.pallas_call(
        mm_body,
        grid=(B // _TILE_B,),
        in_specs=[
            pl.BlockSpec((_TILE_B, L), lambda i: (i, 0)),
            pl.BlockSpec((L, L), lambda i: (0, 0)),
        ],
        out_specs=pl.BlockSpec((_TILE_B, L), lambda i: (i, 0)),
        out_shape=jax.ShapeDtypeStruct((B, L), jnp.float32),
    )(u, r)


def kernel(user, rsid, users_table, rsids_table):
    B = user.shape[0]
    L = rsids_table.shape[1]
    u, r = _sc_gather(user, rsid, users_table, rsids_table, B, L)
    return _tc_matmul(u, r, B, L)


# speedup vs baseline: 1.0533x; 1.0014x over previous
import functools

import jax
import jax.numpy as jnp
from jax import lax
from jax.experimental import pallas as pl
from jax.experimental.pallas import tpu as pltpu
from jax.experimental.pallas import tpu_sc as plsc

_NC, _NS = 2, 16
_NW = _NC * _NS
_STREAM = 256
_TILE_B = 8192


def _sc_gather(user, rsid, users_table, rsids_table, B, L):
    b_per_w = B // _NW
    n_streams = b_per_w // _STREAM
    r_per_w = L // _NS

    mesh = plsc.VectorSubcoreMesh(core_axis_name="c", subcore_axis_name="s")

    @functools.partial(
        pl.kernel,
        out_type=(
            jax.ShapeDtypeStruct((B, L), jnp.float32),
            jax.ShapeDtypeStruct((L, L), jnp.float32),
        ),
        mesh=mesh,
        scratch_types=(
            [
                pltpu.VMEM((b_per_w,), jnp.int32),
                pltpu.VMEM((b_per_w, L), jnp.float32),
                pltpu.VMEM((r_per_w,), jnp.int32),
                pltpu.VMEM((r_per_w, L), jnp.float32),
                pltpu.SemaphoreType.DMA,
                pltpu.SemaphoreType.DMA,
            ]
            + [pltpu.SemaphoreType.DMA for _ in range(n_streams)]
        ),
    )
    def gather_kernel(user_hbm, rsid_hbm, utab_hbm, rtab_hbm, u_out, r_out,
                      uidx_v, urows_v, ridx_v, rrows_v, rsem, wsem, *gsems):
        wid = lax.axis_index("s") * _NC + lax.axis_index("c")
        base = wid * b_per_w

        pltpu.sync_copy(user_hbm.at[pl.ds(base, b_per_w)], uidx_v)
        gathers = [
            pltpu.async_copy(
                utab_hbm.at[uidx_v.at[pl.ds(j * _STREAM, _STREAM)]],
                urows_v.at[pl.ds(j * _STREAM, _STREAM)],
                gsems[j],
            )
            for j in range(n_streams)
        ]
        writes = []
        for j in range(n_streams):
            gathers[j].wait()
            writes.append(pltpu.async_copy(
                urows_v.at[pl.ds(j * _STREAM, _STREAM)],
                u_out.at[pl.ds(base + j * _STREAM, _STREAM)],
                wsem,
            ))

        @pl.when(wid < _NS)
        def _():
            pltpu.sync_copy(rsid_hbm.at[pl.ds(wid * r_per_w, r_per_w)],
                            ridx_v)
            pltpu.async_copy(rtab_hbm.at[ridx_v], rrows_v, rsem).wait()
            pltpu.sync_copy(rrows_v, r_out.at[pl.ds(wid * r_per_w, r_per_w)])

        for c in writes:
            c.wait()

    return gather_kernel(user, rsid, users_table, rsids_table)


def _tc_matmul(u, r, B, L):
    def mm_body(u_ref, r_ref, o_ref):
        o_ref[...] = jnp.dot(u_ref[...], r_ref[...],
                             preferred_element_type=jnp.float32)

    return pl.pallas_call(
        mm_body,
        grid=(B // _TILE_B,),
        in_specs=[
            pl.BlockSpec((_TILE_B, L), lambda i: (i, 0)),
            pl.BlockSpec((L, L), lambda i: (0, 0)),
        ],
        out_specs=pl.BlockSpec((_TILE_B, L), lambda i: (i, 0)),
        out_shape=jax.ShapeDtypeStruct((B, L), jnp.float32),
    )(u, r)


def kernel(user, rsid, users_table, rsids_table):
    B = user.shape[0]
    L = rsids_table.shape[1]
    u, r = _sc_gather(user, rsid, users_table, rsids_table, B, L)
    return _tc_matmul(u, r, B, L)


# speedup vs baseline: 1.0546x; 1.0013x over previous
import functools

import jax
import jax.numpy as jnp
from jax import lax
from jax.experimental import pallas as pl
from jax.experimental.pallas import tpu as pltpu
from jax.experimental.pallas import tpu_sc as plsc

_NC, _NS = 2, 16
_NW = _NC * _NS
_STREAM = 128
_TILE_B = 8192


def _sc_gather(user, rsid, users_table, rsids_table, B, L):
    b_per_w = B // _NW
    n_streams = b_per_w // _STREAM
    r_per_w = L // _NS

    mesh = plsc.VectorSubcoreMesh(core_axis_name="c", subcore_axis_name="s")

    @functools.partial(
        pl.kernel,
        out_type=(
            jax.ShapeDtypeStruct((B, L), jnp.float32),
            jax.ShapeDtypeStruct((L, L), jnp.float32),
        ),
        mesh=mesh,
        scratch_types=(
            [
                pltpu.VMEM((b_per_w,), jnp.int32),
                pltpu.VMEM((b_per_w, L), jnp.float32),
                pltpu.VMEM((r_per_w,), jnp.int32),
                pltpu.VMEM((r_per_w, L), jnp.float32),
                pltpu.SemaphoreType.DMA,
                pltpu.SemaphoreType.DMA,
            ]
            + [pltpu.SemaphoreType.DMA for _ in range(n_streams)]
        ),
    )
    def gather_kernel(user_hbm, rsid_hbm, utab_hbm, rtab_hbm, u_out, r_out,
                      uidx_v, urows_v, ridx_v, rrows_v, rsem, wsem, *gsems):
        wid = lax.axis_index("s") * _NC + lax.axis_index("c")
        base = wid * b_per_w

        pltpu.sync_copy(user_hbm.at[pl.ds(base, b_per_w)], uidx_v)
        gathers = [
            pltpu.async_copy(
                utab_hbm.at[uidx_v.at[pl.ds(j * _STREAM, _STREAM)]],
                urows_v.at[pl.ds(j * _STREAM, _STREAM)],
                gsems[j],
            )
            for j in range(n_streams)
        ]
        writes = []
        for j in range(n_streams):
            gathers[j].wait()
            writes.append(pltpu.async_copy(
                urows_v.at[pl.ds(j * _STREAM, _STREAM)],
                u_out.at[pl.ds(base + j * _STREAM, _STREAM)],
                wsem,
            ))

        @pl.when(wid < _NS)
        def _():
            pltpu.sync_copy(rsid_hbm.at[pl.ds(wid * r_per_w, r_per_w)],
                            ridx_v)
            pltpu.async_copy(rtab_hbm.at[ridx_v], rrows_v, rsem).wait()
            pltpu.sync_copy(rrows_v, r_out.at[pl.ds(wid * r_per_w, r_per_w)])

        for c in writes:
            c.wait()

    return gather_kernel(user, rsid, users_table, rsids_table)


def _tc_matmul(u, r, B, L):
    def mm_body(u_ref, r_ref, o_ref):
        o_ref[...] = jnp.dot(u_ref[...], r_ref[...],
                             preferred_element_type=jnp.float32)

    return pl.pallas_call(
        mm_body,
        grid=(B // _TILE_B,),
        in_specs=[
            pl.BlockSpec((_TILE_B, L), lambda i: (i, 0)),
            pl.BlockSpec((L, L), lambda i: (0, 0)),
        ],
        out_specs=pl.BlockSpec((_TILE_B, L), lambda i: (i, 0)),
        out_shape=jax.ShapeDtypeStruct((B, L), jnp.float32),
    )(u, r)


def kernel(user, rsid, users_table, rsids_table):
    B = user.shape[0]
    L = rsids_table.shape[1]
    u, r = _sc_gather(user, rsid, users_table, rsids_table, B, L)
    return _tc_matmul(u, r, B, L)
